# baseline (device time: 5929 ns/iter reference)
import jax
import jax.numpy as jnp
from jax import lax
from jax.experimental import pallas as pl
from jax.experimental.pallas import tpu as pltpu

EPS = 1e-5
Y_SIZE = 2


def kernel(x, gamma):
    m, n = x.shape
    n_global = Y_SIZE * n
    blocks = m // 128
    half = m // 2

    def body(x_hbm, g_hbm, out_hbm,
             xv_ref, g_ref, out_vmem, partial_ref, recv_ref,
             load_sems, out_sems, send_sem, recv_sem):
        my_x = lax.axis_index("x")
        my_y = lax.axis_index("y")
        nbr = (my_x, 1 - my_y)

        cp_x0 = pltpu.make_async_copy(
            x_hbm.at[pl.ds(0, half), :], xv_ref.at[pl.ds(0, half), :],
            load_sems.at[0])
        cp_x0.start()
        cp_x1 = pltpu.make_async_copy(
            x_hbm.at[pl.ds(half, half), :], xv_ref.at[pl.ds(half, half), :],
            load_sems.at[1])
        cp_x1.start()
        cp_g = pltpu.make_async_copy(g_hbm, g_ref, load_sems.at[2])
        cp_g.start()

        barrier_sem = pltpu.get_barrier_semaphore()
        pl.semaphore_signal(
            barrier_sem, inc=1, device_id=nbr,
            device_id_type=pl.DeviceIdType.MESH,
        )

        hb = blocks // 2
        cp_x0.wait()
        x3a = xv_ref[pl.ds(0, half), :].reshape(hb, 128, n)
        partial_ref[0:hb, :] = jnp.sum(x3a * x3a, axis=2)
        cp_x1.wait()
        x3b = xv_ref[pl.ds(half, half), :].reshape(hb, 128, n)
        partial_ref[hb:blocks, :] = jnp.sum(x3b * x3b, axis=2)

        pl.semaphore_wait(barrier_sem, 1)

        rdma = pltpu.make_async_remote_copy(
            src_ref=partial_ref,
            dst_ref=recv_ref,
            send_sem=send_sem,
            recv_sem=recv_sem,
            device_id=nbr,
            device_id_type=pl.DeviceIdType.MESH,
        )
        rdma.start()

        cp_g.wait()
        gv = g_ref[:, :]
        out_vmem[:, :] = gv * xv_ref[:, :]

        rdma.wait()

        total = partial_ref[:, :] + recv_ref[:, :]
        inv = lax.rsqrt(total / n_global + EPS)
        o3a = out_vmem[pl.ds(0, half), :].reshape(hb, 128, n)
        out_vmem[pl.ds(0, half), :] = (
            o3a * inv[0:hb, :, None]).reshape(half, n)
        cp_o0 = pltpu.make_async_copy(
            out_vmem.at[pl.ds(0, half), :], out_hbm.at[pl.ds(0, half), :],
            out_sems.at[0])
        cp_o0.start()

        o3b = out_vmem[pl.ds(half, half), :].reshape(hb, 128, n)
        out_vmem[pl.ds(half, half), :] = (
            o3b * inv[hb:blocks, :, None]).reshape(half, n)
        cp_o1 = pltpu.make_async_copy(
            out_vmem.at[pl.ds(half, half), :], out_hbm.at[pl.ds(half, half), :],
            out_sems.at[1])
        cp_o1.start()

        cp_o0.wait()
        cp_o1.wait()

    return pl.pallas_call(
        body,
        out_shape=jax.ShapeDtypeStruct((m, n), jnp.float32),
        in_specs=[
            pl.BlockSpec(memory_space=pl.ANY),
            pl.BlockSpec(memory_space=pl.ANY),
        ],
        out_specs=pl.BlockSpec(memory_space=pl.ANY),
        scratch_shapes=[
            pltpu.VMEM((m, n), jnp.float32),
            pltpu.VMEM((1, n), jnp.float32),
            pltpu.VMEM((m, n), jnp.float32),
            pltpu.VMEM((m // 128, 128), jnp.float32),
            pltpu.VMEM((m // 128, 128), jnp.float32),
            pltpu.SemaphoreType.DMA((3,)),
            pltpu.SemaphoreType.DMA((2,)),
            pltpu.SemaphoreType.DMA,
            pltpu.SemaphoreType.DMA,
        ],
        compiler_params=pltpu.CompilerParams(collective_id=0),
    )(
        pltpu.with_memory_space_constraint(x, pltpu.MemorySpace.HBM),
        pltpu.with_memory_space_constraint(
            gamma.reshape(1, n), pltpu.MemorySpace.HBM
        ),
    )
